# baseline (speedup 1.0000x reference)
import functools

import jax
import jax.numpy as jnp
from jax import lax
from jax.experimental import pallas as pl
from jax.experimental.pallas import tpu as pltpu
from jax.experimental.pallas import tpu_sc as plsc

T = 2048
H = 2048
E = 64
K = 8
I = 512
ISH = 1024
TILE = 64
N = T * K
P = N + E * TILE
NT = P // TILE
RT = 256


def _router_body(x_ref, gw_ref, w_ref, i_ref):
    x = x_ref[...]
    logits = jnp.dot(x, gw_ref[...], preferred_element_type=jnp.float32)
    m = jnp.max(logits, axis=-1, keepdims=True)
    p = jnp.exp(logits - m)
    scores = p / jnp.sum(p, axis=-1, keepdims=True)
    iota = jax.lax.broadcasted_iota(jnp.int32, scores.shape, 1)
    cur = scores
    ws, ids = [], []
    for _ in range(K):
        mk = jnp.max(cur, axis=-1, keepdims=True)
        sel = jnp.min(jnp.where(cur == mk, iota, E), axis=-1, keepdims=True)
        ws.append(mk)
        ids.append(sel)
        cur = jnp.where(iota == sel, -jnp.inf, cur)
    w_ref[...] = jnp.concatenate(ws, axis=-1)
    i_ref[...] = jnp.concatenate(ids, axis=-1)


def _router(xf, gate_weight):
    return pl.pallas_call(
        _router_body,
        grid=(T // RT,),
        in_specs=[
            pl.BlockSpec((RT, H), lambda i: (i, 0)),
            pl.BlockSpec((H, E), lambda i: (0, 0)),
        ],
        out_specs=[
            pl.BlockSpec((RT, K), lambda i: (i, 0)),
            pl.BlockSpec((RT, K), lambda i: (i, 0)),
        ],
        out_shape=[
            jax.ShapeDtypeStruct((T, K), jnp.float32),
            jax.ShapeDtypeStruct((T, K), jnp.int32),
        ],
    )(xf, gate_weight)


NW = 16
CH = N // NW
CR = CH // 128


def _meta_sc(idx_flat, w_flat):
    mesh = plsc.VectorSubcoreMesh(core_axis_name="c", subcore_axis_name="s")

    def _vec_ranks(e_vec):
        lane = lax.iota(jnp.int32, 16)
        r = jnp.zeros((16,), jnp.int32)
        tot = jnp.zeros((16,), jnp.int32)
        for j in range(16):
            s = e_vec[j]
            eq = e_vec == jnp.full((16,), s, jnp.int32)
            r = r + jnp.where(eq & (lane > j), 1, 0)
            tot = tot + jnp.where(eq, 1, 0)
        return r, tot

    @functools.partial(
        pl.kernel,
        mesh=mesh,
        compiler_params=pltpu.CompilerParams(needs_layout_passes=False),
        out_type=[
            jax.ShapeDtypeStruct((P,), jnp.int32),
            jax.ShapeDtypeStruct((P,), jnp.float32),
            jax.ShapeDtypeStruct((N,), jnp.int32),
            jax.ShapeDtypeStruct((NT,), jnp.int32),
        ],
        scratch_types=[
            pltpu.VMEM((CR, 128), jnp.int32),
            pltpu.VMEM((CR, 128), jnp.float32),
            pltpu.VMEM((CR, 128), jnp.int32),
            pltpu.VMEM((CR, 128), jnp.int32),
            pltpu.VMEM((128,), jnp.int32),
            pltpu.VMEM((128,), jnp.int32),
            pltpu.VMEM((NW * E,), jnp.int32),
            pltpu.VMEM((E // 16, 16), jnp.int32),
            pltpu.VMEM((NT,), jnp.int32),
            pltpu.VMEM((CR, 128), jnp.int32),
            pltpu.VMEM((CR, 128), jnp.int32),
            pltpu.VMEM_SHARED((NW * E,), jnp.int32),
            pltpu.SemaphoreType.DMA,
        ],
    )
    def k(idx_hbm, wv_hbm, tok_out, ws_out, pos_out, te_out,
          e_buf, w_buf, tok_buf, dest_buf, hist, cur, grid_buf, pcum_buf,
          te_buf, r_buf, tot_buf, sh_hist, sem):
        cid = lax.axis_index("c")
        sid = lax.axis_index("s")

        @pl.when(cid == 0)
        def _():
            base = sid * CH
            for j in range(CR):
                pltpu.sync_copy(idx_hbm.at[pl.ds(base + j * 128, 128)],
                                e_buf.at[j])
                pltpu.sync_copy(wv_hbm.at[pl.ds(base + j * 128, 128)],
                                w_buf.at[j])
            for j in range(E // 16):
                hist[pl.ds(j * 16, 16)] = jnp.zeros((16,), jnp.int32)

            def h_body(j, c):
                for q in range(8):
                    e_vec = e_buf[j, pl.ds(q * 16, 16)]
                    r, tot = _vec_ranks(e_vec)
                    r_buf[j, pl.ds(q * 16, 16)] = r
                    tot_buf[j, pl.ds(q * 16, 16)] = tot
                    h = plsc.load_gather(hist, [e_vec])
                    plsc.store_scatter(hist, [e_vec], h + tot)
                return c
            lax.fori_loop(0, CR, h_body, 0)

            pltpu.sync_copy(hist.at[pl.ds(0, E)], sh_hist.at[pl.ds(sid * E, E)])
            plsc.subcore_barrier()
            pltpu.sync_copy(sh_hist, grid_buf)

            carry = jnp.int32(0)
            for j in range(E // 16):
                tot = jnp.zeros((16,), jnp.int32)
                pre = jnp.zeros((16,), jnp.int32)
                for wkr in range(NW):
                    row = grid_buf[pl.ds(wkr * E + j * 16, 16)]
                    tot = tot + row
                    take = jnp.full((16,), wkr, jnp.int32) < sid
                    pre = pre + jnp.where(take, row, 0)
                pc = ((tot + (TILE - 1)) >> 6) << 6
                inc = plsc.cumsum(pc)
                pcum_buf[j, :] = inc + carry
                cur[pl.ds(j * 16, 16)] = (inc - pc + carry) + pre
                carry = carry + jnp.sum(pc)

            def d_body(j, c):
                for q in range(8):
                    e_vec = e_buf[j, pl.ds(q * 16, 16)]
                    r = r_buf[j, pl.ds(q * 16, 16)]
                    tot = tot_buf[j, pl.ds(q * 16, 16)]
                    d0 = plsc.load_gather(cur, [e_vec])
                    plsc.store_scatter(cur, [e_vec], d0 + tot)
                    dest_buf[j, pl.ds(q * 16, 16)] = d0 + r
                    gidx = base + j * 128 + q * 16 + lax.iota(jnp.int32, 16)
                    tok_buf[j, pl.ds(q * 16, 16)] = gidx >> 3
                return c
            lax.fori_loop(0, CR, d_body, 0)

            for j in range(CR):
                pltpu.sync_copy(dest_buf.at[j],
                                pos_out.at[pl.ds(base + j * 128, 128)])
            cps = []
            for j in range(CR):
                cps.append(pltpu.async_copy(
                    tok_buf.at[j], tok_out.at[dest_buf.at[j]], sem))
                cps.append(pltpu.async_copy(
                    w_buf.at[j], ws_out.at[dest_buf.at[j]], sem))
            for c in cps:
                c.wait()

            @pl.when(sid == 0)
            def _tile_e():
                def t_body(jj, cnts):
                    pv = pcum_buf[jj, :]
                    newc = list(cnts)
                    for l in range(16):
                        sv = jnp.full((16,), pv[l], jnp.int32)
                        for r in range(NT // 16):
                            ts = (r * 16 + lax.iota(jnp.int32, 16)) * TILE
                            newc[r] = newc[r] + jnp.where(sv <= ts, 1, 0)
                    return tuple(newc)
                cnts = lax.fori_loop(
                    0, E // 16, t_body,
                    tuple(jnp.zeros((16,), jnp.int32)
                          for _ in range(NT // 16)))
                for r in range(NT // 16):
                    te_buf[pl.ds(r * 16, 16)] = cnts[r]
                pltpu.sync_copy(te_buf, te_out)

    return k(idx_flat, w_flat)


def _route_metadata(topk_idx, topk_w):
    flat_e = topk_idx.reshape(-1)
    flat_w = topk_w.reshape(-1)
    order = jnp.argsort(flat_e, stable=True)
    sorted_e = flat_e[order]
    counts = jnp.sum(
        flat_e[:, None] == jnp.arange(E, dtype=jnp.int32)[None, :], axis=0
    ).astype(jnp.int32)
    csum = jnp.cumsum(counts)
    starts = csum - counts
    pcounts = ((counts + TILE - 1) // TILE) * TILE
    pcum = jnp.cumsum(pcounts)
    pstarts = pcum - pcounts
    rank = jnp.arange(N, dtype=jnp.int32) - starts[sorted_e]
    dest = pstarts[sorted_e] + rank
    pos_flat = jnp.zeros((N,), jnp.int32).at[order].set(dest, unique_indices=True)
    slots = jnp.arange(P, dtype=jnp.int32)
    slot_e = jnp.searchsorted(pcum, slots, side="right").astype(jnp.int32)
    slot_ec = jnp.minimum(slot_e, E - 1)
    off = slots - pstarts[slot_ec]
    valid = (slot_e < E) & (off < counts[slot_ec])
    s_idx = jnp.minimum(starts[slot_ec] + off, N - 1)
    src = order[s_idx]
    token_slot = jnp.where(valid, src // K, 0)
    w_slot = jnp.where(valid, flat_w[src], 0.0)
    tile_starts = jnp.arange(NT, dtype=jnp.int32) * TILE
    tile_e = jnp.minimum(
        jnp.searchsorted(pcum, tile_starts, side="right"), E - 1
    ).astype(jnp.int32)
    return token_slot, w_slot, pos_flat, tile_e


def _ffn_body(te_ref, x_ref, gu_ref, dw_ref, w_ref, o_ref):
    @pl.when(te_ref[pl.program_id(0)] < E)
    def _():
        x = x_ref[...].astype(jnp.float32)
        gu = jnp.dot(x, gu_ref[0], preferred_element_type=jnp.float32,
                     precision=jax.lax.Precision.DEFAULT)
        g = gu[:, :I]
        u = gu[:, I:]
        h = g * jax.nn.sigmoid(g) * u
        y = jnp.dot(h, dw_ref[0], preferred_element_type=jnp.float32,
                    precision=jax.lax.Precision.DEFAULT)
        o_ref[...] = (y * w_ref[...]).astype(jnp.bfloat16)


def _grouped_ffn(tile_e, x_sorted, gate_up_proj, down_proj, w_col):
    grid_spec = pltpu.PrefetchScalarGridSpec(
        num_scalar_prefetch=1,
        grid=(NT,),
        in_specs=[
            pl.BlockSpec((TILE, H), lambda i, te: (i, 0)),
            pl.BlockSpec((1, H, 2 * I),
                         lambda i, te: (jnp.minimum(te[i], E - 1), 0, 0)),
            pl.BlockSpec((1, I, H),
                         lambda i, te: (jnp.minimum(te[i], E - 1), 0, 0)),
            pl.BlockSpec((TILE, 1), lambda i, te: (i, 0)),
        ],
        out_specs=pl.BlockSpec((TILE, H), lambda i, te: (i, 0)),
    )
    return pl.pallas_call(
        _ffn_body,
        grid_spec=grid_spec,
        out_shape=jax.ShapeDtypeStruct((P, H), jnp.bfloat16),
    )(tile_e, x_sorted, gate_up_proj, down_proj, w_col)


def _shared_body(x_ref, gw_ref, uw_ref, dw_ref, o_ref):
    x = x_ref[...]
    g = jnp.dot(x, gw_ref[...], preferred_element_type=jnp.float32,
                precision=jax.lax.Precision.DEFAULT)
    u = jnp.dot(x, uw_ref[...], preferred_element_type=jnp.float32,
                precision=jax.lax.Precision.DEFAULT)
    h = g * jax.nn.sigmoid(g) * u
    y = jnp.dot(h, dw_ref[...], preferred_element_type=jnp.float32,
                precision=jax.lax.Precision.DEFAULT)
    o_ref[...] = y


def _shared(xf, sgw, suw, sdw):
    return pl.pallas_call(
        _shared_body,
        grid=(T // RT,),
        in_specs=[
            pl.BlockSpec((RT, H), lambda i: (i, 0)),
            pl.BlockSpec((H, ISH), lambda i: (0, 0)),
            pl.BlockSpec((H, ISH), lambda i: (0, 0)),
            pl.BlockSpec((ISH, H), lambda i: (0, 0)),
        ],
        out_specs=pl.BlockSpec((RT, H), lambda i: (i, 0)),
        out_shape=jax.ShapeDtypeStruct((T, H), jnp.float32),
    )(xf, sgw, suw, sdw)


def kernel(hidden_states, gate_weight, gate_up_proj, down_proj,
           shared_gate_w, shared_up_w, shared_down_w):
    xf = hidden_states.reshape(T, H)
    topk_w, topk_idx = _router(xf, gate_weight)
    token_slot, w_slot, pos_flat, tile_e = _meta_sc(
        topk_idx.reshape(-1), topk_w.reshape(-1))
    shared_out = _shared(xf, shared_gate_w, shared_up_w, shared_down_w)
    x_sorted = jnp.take(xf.astype(jnp.bfloat16), token_slot, axis=0,
                        mode="clip")
    y_pad = _grouped_ffn(tile_e, x_sorted, gate_up_proj, down_proj,
                         w_slot[:, None])
    routed = jnp.sum(
        y_pad[pos_flat.reshape(T, K)].astype(jnp.float32), axis=1)
    out = routed + shared_out
    return out.reshape(hidden_states.shape)


# speedup vs baseline: 1.1425x; 1.1425x over previous
import functools

import jax
import jax.numpy as jnp
from jax import lax
from jax.experimental import pallas as pl
from jax.experimental.pallas import tpu as pltpu
from jax.experimental.pallas import tpu_sc as plsc

T = 2048
H = 2048
E = 64
K = 8
I = 512
ISH = 1024
TILE = 128
N = T * K
P = N + E * TILE
NT = P // TILE
RT = 256


def _router_body(x_ref, gw_ref, w_ref, i_ref):
    x = x_ref[...]
    logits = jnp.dot(x, gw_ref[...], preferred_element_type=jnp.float32)
    m = jnp.max(logits, axis=-1, keepdims=True)
    p = jnp.exp(logits - m)
    scores = p / jnp.sum(p, axis=-1, keepdims=True)
    iota = jax.lax.broadcasted_iota(jnp.int32, scores.shape, 1)
    cur = scores
    ws, ids = [], []
    for _ in range(K):
        mk = jnp.max(cur, axis=-1, keepdims=True)
        sel = jnp.min(jnp.where(cur == mk, iota, E), axis=-1, keepdims=True)
        ws.append(mk)
        ids.append(sel)
        cur = jnp.where(iota == sel, -jnp.inf, cur)
    w_ref[...] = jnp.concatenate(ws, axis=-1)
    i_ref[...] = jnp.concatenate(ids, axis=-1)


def _router(xf, gate_weight):
    return pl.pallas_call(
        _router_body,
        grid=(T // RT,),
        in_specs=[
            pl.BlockSpec((RT, H), lambda i: (i, 0)),
            pl.BlockSpec((H, E), lambda i: (0, 0)),
        ],
        out_specs=[
            pl.BlockSpec((RT, K), lambda i: (i, 0)),
            pl.BlockSpec((RT, K), lambda i: (i, 0)),
        ],
        out_shape=[
            jax.ShapeDtypeStruct((T, K), jnp.float32),
            jax.ShapeDtypeStruct((T, K), jnp.int32),
        ],
    )(xf, gate_weight)


NW = 16
CH = N // NW
CR = CH // 128


def _meta_sc(idx_flat, w_flat):
    mesh = plsc.VectorSubcoreMesh(core_axis_name="c", subcore_axis_name="s")

    def _vec_ranks(e_vec):
        lane = lax.iota(jnp.int32, 16)
        r = jnp.zeros((16,), jnp.int32)
        tot = jnp.zeros((16,), jnp.int32)
        for j in range(16):
            s = e_vec[j]
            eq = e_vec == jnp.full((16,), s, jnp.int32)
            r = r + jnp.where(eq & (lane > j), 1, 0)
            tot = tot + jnp.where(eq, 1, 0)
        return r, tot

    @functools.partial(
        pl.kernel,
        mesh=mesh,
        compiler_params=pltpu.CompilerParams(needs_layout_passes=False),
        out_type=[
            jax.ShapeDtypeStruct((P,), jnp.int32),
            jax.ShapeDtypeStruct((P,), jnp.float32),
            jax.ShapeDtypeStruct((N,), jnp.int32),
            jax.ShapeDtypeStruct((NT,), jnp.int32),
        ],
        scratch_types=[
            pltpu.VMEM((CR, 128), jnp.int32),
            pltpu.VMEM((CR, 128), jnp.float32),
            pltpu.VMEM((CR, 128), jnp.int32),
            pltpu.VMEM((CR, 128), jnp.int32),
            pltpu.VMEM((128,), jnp.int32),
            pltpu.VMEM((128,), jnp.int32),
            pltpu.VMEM((NW * E,), jnp.int32),
            pltpu.VMEM((E // 16, 16), jnp.int32),
            pltpu.VMEM((NT,), jnp.int32),
            pltpu.VMEM((CR, 128), jnp.int32),
            pltpu.VMEM((CR, 128), jnp.int32),
            pltpu.VMEM_SHARED((NW * E,), jnp.int32),
            pltpu.SemaphoreType.DMA,
        ],
    )
    def k(idx_hbm, wv_hbm, tok_out, ws_out, pos_out, te_out,
          e_buf, w_buf, tok_buf, dest_buf, hist, cur, grid_buf, pcum_buf,
          te_buf, r_buf, tot_buf, sh_hist, sem):
        cid = lax.axis_index("c")
        sid = lax.axis_index("s")

        @pl.when(cid == 0)
        def _():
            base = sid * CH
            for j in range(CR):
                pltpu.sync_copy(idx_hbm.at[pl.ds(base + j * 128, 128)],
                                e_buf.at[j])
                pltpu.sync_copy(wv_hbm.at[pl.ds(base + j * 128, 128)],
                                w_buf.at[j])
            for j in range(E // 16):
                hist[pl.ds(j * 16, 16)] = jnp.zeros((16,), jnp.int32)

            def h_body(j, c):
                for q in range(8):
                    e_vec = e_buf[j, pl.ds(q * 16, 16)]
                    r, tot = _vec_ranks(e_vec)
                    r_buf[j, pl.ds(q * 16, 16)] = r
                    tot_buf[j, pl.ds(q * 16, 16)] = tot
                    h = plsc.load_gather(hist, [e_vec])
                    plsc.store_scatter(hist, [e_vec], h + tot)
                return c
            lax.fori_loop(0, CR, h_body, 0)

            pltpu.sync_copy(hist.at[pl.ds(0, E)], sh_hist.at[pl.ds(sid * E, E)])
            plsc.subcore_barrier()
            pltpu.sync_copy(sh_hist, grid_buf)

            carry = jnp.int32(0)
            for j in range(E // 16):
                tot = jnp.zeros((16,), jnp.int32)
                pre = jnp.zeros((16,), jnp.int32)
                for wkr in range(NW):
                    row = grid_buf[pl.ds(wkr * E + j * 16, 16)]
                    tot = tot + row
                    take = jnp.full((16,), wkr, jnp.int32) < sid
                    pre = pre + jnp.where(take, row, 0)
                pc = ((tot + (TILE - 1)) >> 7) << 7
                inc = plsc.cumsum(pc)
                pcum_buf[j, :] = inc + carry
                cur[pl.ds(j * 16, 16)] = (inc - pc + carry) + pre
                carry = carry + jnp.sum(pc)

            def d_body(j, c):
                for q in range(8):
                    e_vec = e_buf[j, pl.ds(q * 16, 16)]
                    r = r_buf[j, pl.ds(q * 16, 16)]
                    tot = tot_buf[j, pl.ds(q * 16, 16)]
                    d0 = plsc.load_gather(cur, [e_vec])
                    plsc.store_scatter(cur, [e_vec], d0 + tot)
                    dest_buf[j, pl.ds(q * 16, 16)] = d0 + r
                    gidx = base + j * 128 + q * 16 + lax.iota(jnp.int32, 16)
                    tok_buf[j, pl.ds(q * 16, 16)] = gidx >> 3
                return c
            lax.fori_loop(0, CR, d_body, 0)

            for j in range(CR):
                pltpu.sync_copy(dest_buf.at[j],
                                pos_out.at[pl.ds(base + j * 128, 128)])
            cps = []
            for j in range(CR):
                cps.append(pltpu.async_copy(
                    tok_buf.at[j], tok_out.at[dest_buf.at[j]], sem))
                cps.append(pltpu.async_copy(
                    w_buf.at[j], ws_out.at[dest_buf.at[j]], sem))
            for c in cps:
                c.wait()

            @pl.when(sid == 0)
            def _tile_e():
                def t_body(jj, cnts):
                    pv = pcum_buf[jj, :]
                    newc = list(cnts)
                    for l in range(16):
                        sv = jnp.full((16,), pv[l], jnp.int32)
                        for r in range(NT // 16):
                            ts = (r * 16 + lax.iota(jnp.int32, 16)) * TILE
                            newc[r] = newc[r] + jnp.where(sv <= ts, 1, 0)
                    return tuple(newc)
                cnts = lax.fori_loop(
                    0, E // 16, t_body,
                    tuple(jnp.zeros((16,), jnp.int32)
                          for _ in range(NT // 16)))
                for r in range(NT // 16):
                    te_buf[pl.ds(r * 16, 16)] = cnts[r]
                pltpu.sync_copy(te_buf, te_out)

    return k(idx_flat, w_flat)


def _route_metadata(topk_idx, topk_w):
    flat_e = topk_idx.reshape(-1)
    flat_w = topk_w.reshape(-1)
    order = jnp.argsort(flat_e, stable=True)
    sorted_e = flat_e[order]
    counts = jnp.sum(
        flat_e[:, None] == jnp.arange(E, dtype=jnp.int32)[None, :], axis=0
    ).astype(jnp.int32)
    csum = jnp.cumsum(counts)
    starts = csum - counts
    pcounts = ((counts + TILE - 1) // TILE) * TILE
    pcum = jnp.cumsum(pcounts)
    pstarts = pcum - pcounts
    rank = jnp.arange(N, dtype=jnp.int32) - starts[sorted_e]
    dest = pstarts[sorted_e] + rank
    pos_flat = jnp.zeros((N,), jnp.int32).at[order].set(dest, unique_indices=True)
    slots = jnp.arange(P, dtype=jnp.int32)
    slot_e = jnp.searchsorted(pcum, slots, side="right").astype(jnp.int32)
    slot_ec = jnp.minimum(slot_e, E - 1)
    off = slots - pstarts[slot_ec]
    valid = (slot_e < E) & (off < counts[slot_ec])
    s_idx = jnp.minimum(starts[slot_ec] + off, N - 1)
    src = order[s_idx]
    token_slot = jnp.where(valid, src // K, 0)
    w_slot = jnp.where(valid, flat_w[src], 0.0)
    tile_starts = jnp.arange(NT, dtype=jnp.int32) * TILE
    tile_e = jnp.minimum(
        jnp.searchsorted(pcum, tile_starts, side="right"), E - 1
    ).astype(jnp.int32)
    return token_slot, w_slot, pos_flat, tile_e


def _ffn_body(te_ref, x_ref, gu_ref, dw_ref, w_ref, o_ref):
    @pl.when(te_ref[pl.program_id(0)] < E)
    def _():
        x = x_ref[...].astype(jnp.float32)
        gu = jnp.dot(x, gu_ref[0], preferred_element_type=jnp.float32,
                     precision=jax.lax.Precision.DEFAULT)
        g = gu[:, :I]
        u = gu[:, I:]
        h = g * jax.nn.sigmoid(g) * u
        y = jnp.dot(h, dw_ref[0], preferred_element_type=jnp.float32,
                    precision=jax.lax.Precision.DEFAULT)
        o_ref[...] = (y * w_ref[...]).astype(jnp.bfloat16)


def _grouped_ffn(tile_e, x_sorted, gate_up_proj, down_proj, w_col):
    grid_spec = pltpu.PrefetchScalarGridSpec(
        num_scalar_prefetch=1,
        grid=(NT,),
        in_specs=[
            pl.BlockSpec((TILE, H), lambda i, te: (i, 0)),
            pl.BlockSpec((1, H, 2 * I),
                         lambda i, te: (jnp.minimum(te[i], E - 1), 0, 0)),
            pl.BlockSpec((1, I, H),
                         lambda i, te: (jnp.minimum(te[i], E - 1), 0, 0)),
            pl.BlockSpec((TILE, 1), lambda i, te: (i, 0)),
        ],
        out_specs=pl.BlockSpec((TILE, H), lambda i, te: (i, 0)),
    )
    return pl.pallas_call(
        _ffn_body,
        grid_spec=grid_spec,
        out_shape=jax.ShapeDtypeStruct((P, H), jnp.bfloat16),
    )(tile_e, x_sorted, gate_up_proj, down_proj, w_col)


def _shared_body(x_ref, gw_ref, uw_ref, dw_ref, o_ref):
    x = x_ref[...]
    g = jnp.dot(x, gw_ref[...], preferred_element_type=jnp.float32,
                precision=jax.lax.Precision.DEFAULT)
    u = jnp.dot(x, uw_ref[...], preferred_element_type=jnp.float32,
                precision=jax.lax.Precision.DEFAULT)
    h = g * jax.nn.sigmoid(g) * u
    y = jnp.dot(h, dw_ref[...], preferred_element_type=jnp.float32,
                precision=jax.lax.Precision.DEFAULT)
    o_ref[...] = y


def _shared(xf, sgw, suw, sdw):
    return pl.pallas_call(
        _shared_body,
        grid=(T // RT,),
        in_specs=[
            pl.BlockSpec((RT, H), lambda i: (i, 0)),
            pl.BlockSpec((H, ISH), lambda i: (0, 0)),
            pl.BlockSpec((H, ISH), lambda i: (0, 0)),
            pl.BlockSpec((ISH, H), lambda i: (0, 0)),
        ],
        out_specs=pl.BlockSpec((RT, H), lambda i: (i, 0)),
        out_shape=jax.ShapeDtypeStruct((T, H), jnp.float32),
    )(xf, sgw, suw, sdw)


def kernel(hidden_states, gate_weight, gate_up_proj, down_proj,
           shared_gate_w, shared_up_w, shared_down_w):
    xf = hidden_states.reshape(T, H)
    topk_w, topk_idx = _router(xf, gate_weight)
    token_slot, w_slot, pos_flat, tile_e = _meta_sc(
        topk_idx.reshape(-1), topk_w.reshape(-1))
    shared_out = _shared(xf, shared_gate_w, shared_up_w, shared_down_w)
    x_sorted = jnp.take(xf.astype(jnp.bfloat16), token_slot, axis=0,
                        mode="clip")
    y_pad = _grouped_ffn(tile_e, x_sorted, gate_up_proj, down_proj,
                         w_slot[:, None])
    routed = jnp.sum(
        y_pad[pos_flat.reshape(T, K)].astype(jnp.float32), axis=1)
    out = routed + shared_out
    return out.reshape(hidden_states.shape)


# speedup vs baseline: 1.1437x; 1.0010x over previous
import functools

import jax
import jax.numpy as jnp
from jax import lax
from jax.experimental import pallas as pl
from jax.experimental.pallas import tpu as pltpu
from jax.experimental.pallas import tpu_sc as plsc

T = 2048
H = 2048
E = 64
K = 8
I = 512
ISH = 1024
TILE = 128
N = T * K
P = N + E * TILE
NT = P // TILE
RT = 256


def _router_body(x_ref, gw_ref, w_ref, i_ref):
    x = x_ref[...]
    logits = jnp.dot(x, gw_ref[...], preferred_element_type=jnp.float32)
    m = jnp.max(logits, axis=-1, keepdims=True)
    p = jnp.exp(logits - m)
    scores = p / jnp.sum(p, axis=-1, keepdims=True)
    iota = jax.lax.broadcasted_iota(jnp.int32, scores.shape, 1)
    cur = scores
    ws, ids = [], []
    for _ in range(K):
        mk = jnp.max(cur, axis=-1, keepdims=True)
        sel = jnp.min(jnp.where(cur == mk, iota, E), axis=-1, keepdims=True)
        ws.append(mk)
        ids.append(sel)
        cur = jnp.where(iota == sel, -jnp.inf, cur)
    w_ref[...] = jnp.concatenate(ws, axis=-1)
    i_ref[...] = jnp.concatenate(ids, axis=-1)


def _router(xf, gate_weight):
    return pl.pallas_call(
        _router_body,
        grid=(T // RT,),
        in_specs=[
            pl.BlockSpec((RT, H), lambda i: (i, 0)),
            pl.BlockSpec((H, E), lambda i: (0, 0)),
        ],
        out_specs=[
            pl.BlockSpec((RT, K), lambda i: (i, 0)),
            pl.BlockSpec((RT, K), lambda i: (i, 0)),
        ],
        out_shape=[
            jax.ShapeDtypeStruct((T, K), jnp.float32),
            jax.ShapeDtypeStruct((T, K), jnp.int32),
        ],
    )(xf, gate_weight)


NW = 16
CH = N // NW
CR = CH // 128


def _meta_sc(idx_flat, w_flat):
    mesh = plsc.VectorSubcoreMesh(core_axis_name="c", subcore_axis_name="s")

    def _vec_ranks(e_vec):
        lane = lax.iota(jnp.int32, 16)
        r = jnp.zeros((16,), jnp.int32)
        tot = jnp.zeros((16,), jnp.int32)
        for j in range(16):
            s = e_vec[j]
            eq = e_vec == jnp.full((16,), s, jnp.int32)
            r = r + jnp.where(eq & (lane > j), 1, 0)
            tot = tot + jnp.where(eq, 1, 0)
        return r, tot

    @functools.partial(
        pl.kernel,
        mesh=mesh,
        compiler_params=pltpu.CompilerParams(needs_layout_passes=False),
        out_type=[
            jax.ShapeDtypeStruct((P,), jnp.int32),
            jax.ShapeDtypeStruct((P,), jnp.float32),
            jax.ShapeDtypeStruct((N,), jnp.int32),
            jax.ShapeDtypeStruct((NT,), jnp.int32),
        ],
        scratch_types=[
            pltpu.VMEM((CR, 128), jnp.int32),
            pltpu.VMEM((CR, 128), jnp.float32),
            pltpu.VMEM((CR, 128), jnp.int32),
            pltpu.VMEM((CR, 128), jnp.int32),
            pltpu.VMEM((128,), jnp.int32),
            pltpu.VMEM((128,), jnp.int32),
            pltpu.VMEM((NW * E,), jnp.int32),
            pltpu.VMEM((E // 16, 16), jnp.int32),
            pltpu.VMEM((NT,), jnp.int32),
            pltpu.VMEM((CR, 128), jnp.int32),
            pltpu.VMEM((CR, 128), jnp.int32),
            pltpu.VMEM_SHARED((NW * E,), jnp.int32),
            pltpu.SemaphoreType.DMA,
        ],
    )
    def k(idx_hbm, wv_hbm, tok_out, ws_out, pos_out, te_out,
          e_buf, w_buf, tok_buf, dest_buf, hist, cur, grid_buf, pcum_buf,
          te_buf, r_buf, tot_buf, sh_hist, sem):
        cid = lax.axis_index("c")
        sid = lax.axis_index("s")

        @pl.when(cid == 0)
        def _():
            base = sid * CH
            for j in range(CR):
                pltpu.sync_copy(idx_hbm.at[pl.ds(base + j * 128, 128)],
                                e_buf.at[j])
                pltpu.sync_copy(wv_hbm.at[pl.ds(base + j * 128, 128)],
                                w_buf.at[j])
            for j in range(E // 16):
                hist[pl.ds(j * 16, 16)] = jnp.zeros((16,), jnp.int32)

            def h_body(j, c):
                for q in range(8):
                    e_vec = e_buf[j, pl.ds(q * 16, 16)]
                    r, tot = _vec_ranks(e_vec)
                    r_buf[j, pl.ds(q * 16, 16)] = r
                    tot_buf[j, pl.ds(q * 16, 16)] = tot
                    h = plsc.load_gather(hist, [e_vec])
                    plsc.store_scatter(hist, [e_vec], h + tot)
                return c
            lax.fori_loop(0, CR, h_body, 0)

            pltpu.sync_copy(hist.at[pl.ds(0, E)], sh_hist.at[pl.ds(sid * E, E)])
            plsc.subcore_barrier()
            pltpu.sync_copy(sh_hist, grid_buf)

            carry = jnp.int32(0)
            for j in range(E // 16):
                tot = jnp.zeros((16,), jnp.int32)
                pre = jnp.zeros((16,), jnp.int32)
                for wkr in range(NW):
                    row = grid_buf[pl.ds(wkr * E + j * 16, 16)]
                    tot = tot + row
                    take = jnp.full((16,), wkr, jnp.int32) < sid
                    pre = pre + jnp.where(take, row, 0)
                pc = ((tot + (TILE - 1)) >> 7) << 7
                inc = plsc.cumsum(pc)
                pcum_buf[j, :] = inc + carry
                cur[pl.ds(j * 16, 16)] = (inc - pc + carry) + pre
                carry = carry + jnp.sum(pc)

            def d_body(j, c):
                for q in range(8):
                    e_vec = e_buf[j, pl.ds(q * 16, 16)]
                    r = r_buf[j, pl.ds(q * 16, 16)]
                    tot = tot_buf[j, pl.ds(q * 16, 16)]
                    d0 = plsc.load_gather(cur, [e_vec])
                    plsc.store_scatter(cur, [e_vec], d0 + tot)
                    dest_buf[j, pl.ds(q * 16, 16)] = d0 + r
                    gidx = base + j * 128 + q * 16 + lax.iota(jnp.int32, 16)
                    tok_buf[j, pl.ds(q * 16, 16)] = gidx >> 3
                return c
            lax.fori_loop(0, CR, d_body, 0)

            for j in range(CR):
                pltpu.sync_copy(dest_buf.at[j],
                                pos_out.at[pl.ds(base + j * 128, 128)])
            cps = []
            for j in range(CR):
                cps.append(pltpu.async_copy(
                    tok_buf.at[j], tok_out.at[dest_buf.at[j]], sem))
                cps.append(pltpu.async_copy(
                    w_buf.at[j], ws_out.at[dest_buf.at[j]], sem))
            for c in cps:
                c.wait()

            @pl.when(sid == 0)
            def _tile_e():
                def t_body(jj, cnts):
                    pv = pcum_buf[jj, :]
                    newc = list(cnts)
                    for l in range(16):
                        sv = jnp.full((16,), pv[l], jnp.int32)
                        for r in range(NT // 16):
                            ts = (r * 16 + lax.iota(jnp.int32, 16)) * TILE
                            newc[r] = newc[r] + jnp.where(sv <= ts, 1, 0)
                    return tuple(newc)
                cnts = lax.fori_loop(
                    0, E // 16, t_body,
                    tuple(jnp.zeros((16,), jnp.int32)
                          for _ in range(NT // 16)))
                for r in range(NT // 16):
                    te_buf[pl.ds(r * 16, 16)] = cnts[r]
                pltpu.sync_copy(te_buf, te_out)

    return k(idx_flat, w_flat)


def _route_metadata(topk_idx, topk_w):
    flat_e = topk_idx.reshape(-1)
    flat_w = topk_w.reshape(-1)
    order = jnp.argsort(flat_e, stable=True)
    sorted_e = flat_e[order]
    counts = jnp.sum(
        flat_e[:, None] == jnp.arange(E, dtype=jnp.int32)[None, :], axis=0
    ).astype(jnp.int32)
    csum = jnp.cumsum(counts)
    starts = csum - counts
    pcounts = ((counts + TILE - 1) // TILE) * TILE
    pcum = jnp.cumsum(pcounts)
    pstarts = pcum - pcounts
    rank = jnp.arange(N, dtype=jnp.int32) - starts[sorted_e]
    dest = pstarts[sorted_e] + rank
    pos_flat = jnp.zeros((N,), jnp.int32).at[order].set(dest, unique_indices=True)
    slots = jnp.arange(P, dtype=jnp.int32)
    slot_e = jnp.searchsorted(pcum, slots, side="right").astype(jnp.int32)
    slot_ec = jnp.minimum(slot_e, E - 1)
    off = slots - pstarts[slot_ec]
    valid = (slot_e < E) & (off < counts[slot_ec])
    s_idx = jnp.minimum(starts[slot_ec] + off, N - 1)
    src = order[s_idx]
    token_slot = jnp.where(valid, src // K, 0)
    w_slot = jnp.where(valid, flat_w[src], 0.0)
    tile_starts = jnp.arange(NT, dtype=jnp.int32) * TILE
    tile_e = jnp.minimum(
        jnp.searchsorted(pcum, tile_starts, side="right"), E - 1
    ).astype(jnp.int32)
    return token_slot, w_slot, pos_flat, tile_e


def _ffn_body(te_ref, x_ref, g_ref, u_ref, d1_ref, d2_ref, w_ref, o_ref):
    @pl.when(te_ref[pl.program_id(0)] < E)
    def _():
        x = x_ref[...].astype(jnp.float32)
        g = jnp.dot(x, g_ref[0], preferred_element_type=jnp.float32,
                    precision=jax.lax.Precision.DEFAULT)
        u = jnp.dot(x, u_ref[0], preferred_element_type=jnp.float32,
                    precision=jax.lax.Precision.DEFAULT)
        h = g * jax.nn.sigmoid(g) * u
        y1 = jnp.dot(h[:, :I // 2], d1_ref[0],
                     preferred_element_type=jnp.float32,
                     precision=jax.lax.Precision.DEFAULT)
        y2 = jnp.dot(h[:, I // 2:], d2_ref[0],
                     preferred_element_type=jnp.float32,
                     precision=jax.lax.Precision.DEFAULT)
        o_ref[...] = ((y1 + y2) * w_ref[...]).astype(jnp.bfloat16)


def _grouped_ffn(tile_e, x_sorted, gate_up_proj, down_proj, w_col):
    grid_spec = pltpu.PrefetchScalarGridSpec(
        num_scalar_prefetch=1,
        grid=(NT,),
        in_specs=[
            pl.BlockSpec((TILE, H), lambda i, te: (i, 0)),
            pl.BlockSpec((1, H, I),
                         lambda i, te: (jnp.minimum(te[i], E - 1), 0, 0)),
            pl.BlockSpec((1, H, I),
                         lambda i, te: (jnp.minimum(te[i], E - 1), 0, 1)),
            pl.BlockSpec((1, I // 2, H),
                         lambda i, te: (jnp.minimum(te[i], E - 1), 0, 0)),
            pl.BlockSpec((1, I // 2, H),
                         lambda i, te: (jnp.minimum(te[i], E - 1), 1, 0)),
            pl.BlockSpec((TILE, 1), lambda i, te: (i, 0)),
        ],
        out_specs=pl.BlockSpec((TILE, H), lambda i, te: (i, 0)),
    )
    return pl.pallas_call(
        _ffn_body,
        grid_spec=grid_spec,
        out_shape=jax.ShapeDtypeStruct((P, H), jnp.bfloat16),
    )(tile_e, x_sorted, gate_up_proj, gate_up_proj, down_proj, down_proj,
      w_col)


def _shared_body(x_ref, gw_ref, uw_ref, dw_ref, o_ref):
    x = x_ref[...]
    g = jnp.dot(x, gw_ref[...], preferred_element_type=jnp.float32,
                precision=jax.lax.Precision.DEFAULT)
    u = jnp.dot(x, uw_ref[...], preferred_element_type=jnp.float32,
                precision=jax.lax.Precision.DEFAULT)
    h = g * jax.nn.sigmoid(g) * u
    y = jnp.dot(h, dw_ref[...], preferred_element_type=jnp.float32,
                precision=jax.lax.Precision.DEFAULT)
    o_ref[...] = y


def _shared(xf, sgw, suw, sdw):
    return pl.pallas_call(
        _shared_body,
        grid=(T // RT,),
        in_specs=[
            pl.BlockSpec((RT, H), lambda i: (i, 0)),
            pl.BlockSpec((H, ISH), lambda i: (0, 0)),
            pl.BlockSpec((H, ISH), lambda i: (0, 0)),
            pl.BlockSpec((ISH, H), lambda i: (0, 0)),
        ],
        out_specs=pl.BlockSpec((RT, H), lambda i: (i, 0)),
        out_shape=jax.ShapeDtypeStruct((T, H), jnp.float32),
    )(xf, sgw, suw, sdw)


def kernel(hidden_states, gate_weight, gate_up_proj, down_proj,
           shared_gate_w, shared_up_w, shared_down_w):
    xf = hidden_states.reshape(T, H)
    topk_w, topk_idx = _router(xf, gate_weight)
    token_slot, w_slot, pos_flat, tile_e = _meta_sc(
        topk_idx.reshape(-1), topk_w.reshape(-1))
    shared_out = _shared(xf, shared_gate_w, shared_up_w, shared_down_w)
    x_sorted = jnp.take(xf.astype(jnp.bfloat16), token_slot, axis=0,
                        mode="clip")
    y_pad = _grouped_ffn(tile_e, x_sorted, gate_up_proj, down_proj,
                         w_slot[:, None])
    routed = jnp.sum(
        y_pad[pos_flat.reshape(T, K)].astype(jnp.float32), axis=1)
    out = routed + shared_out
    return out.reshape(hidden_states.shape)
